# Initial kernel scaffold; baseline (speedup 1.0000x reference)
import jax
import jax.numpy as jnp
from jax.experimental import pallas as pl


def kernel(x, edge_index, W0, b0, g0, be0, W1, b1, g1, be1, W2, b2, g2, be2):
    raise NotImplementedError("write your pallas kernel here")


# speedup vs baseline: 8.7822x; 8.7822x over previous
import functools

import jax
import jax.numpy as jnp
from jax import lax
from jax.experimental import pallas as pl
from jax.experimental.pallas import tpu as pltpu
from jax.experimental.pallas import tpu_sc as plsc

N = 10000
E = 320000
D_IN = 128
D_H = 256
HALF = 128
EPS = 1e-5

NC = 2
NS = 16
CHUNK = 128
E_PAD = 157 * 2048
EPT = E_PAD // NS
N_CHUNKS = EPT // CHUNK
ACC_ROWS = 10240
ZR = ACC_ROWS // NS

DEG_EPT = E_PAD // (NC * NS)
DEG_CHUNK = 64
DEG_NCHUNKS = DEG_EPT // DEG_CHUNK


def _sc_mesh():
  return plsc.VectorSubcoreMesh(
      core_axis_name="c", subcore_axis_name="s", num_cores=NC, num_subcores=NS)


@functools.partial(
    pl.kernel,
    out_type=jax.ShapeDtypeStruct((NC, ACC_ROWS, 16), jnp.float32),
    mesh=_sc_mesh(),
    scratch_types=[
        pltpu.VMEM((DEG_CHUNK,), jnp.int32),
        pltpu.VMEM((DEG_CHUNK, 16), jnp.float32),
        pltpu.VMEM_SHARED((ACC_ROWS, 16), jnp.float32),
    ],
)
def _sc_degree(dst_hbm, zero_hbm, out_hbm, didx, ones, acc):
  c = lax.axis_index("c")
  s = lax.axis_index("s")

  pltpu.sync_copy(zero_hbm.at[pl.ds(0, ZR)], acc.at[pl.ds(s * ZR, ZR)])

  @pl.loop(0, DEG_CHUNK)
  def _fill(r):
    ones[r, :] = jnp.full((16,), 1.0, dtype=jnp.float32)

  plsc.subcore_barrier()

  base = (c * NS + s) * DEG_EPT

  @pl.loop(0, DEG_NCHUNKS)
  def _body(g):
    off = base + g * DEG_CHUNK
    pltpu.sync_copy(dst_hbm.at[pl.ds(off, DEG_CHUNK)], didx)
    pltpu.sync_copy(ones, acc.at[didx], add=True)

  plsc.subcore_barrier()
  pltpu.sync_copy(acc.at[pl.ds(s * ZR, ZR)], out_hbm.at[c, pl.ds(s * ZR, ZR)])


@functools.partial(
    pl.kernel,
    out_type=jax.ShapeDtypeStruct((NC * ACC_ROWS, HALF), jnp.float32),
    mesh=_sc_mesh(),
    scratch_types=[
        pltpu.VMEM((CHUNK,), jnp.int32),
        pltpu.VMEM((CHUNK,), jnp.int32),
        pltpu.VMEM((CHUNK, HALF), jnp.float32),
        pltpu.VMEM_SHARED((ACC_ROWS, HALF), jnp.float32),
        pltpu.SemaphoreType.DMA,
    ],
)
def _sc_aggregate(h2_hbm, src_hbm, dst_hbm, zero_hbm, out_hbm,
                  sidx, didx, rows, acc, gsem):
  c = lax.axis_index("c")
  s = lax.axis_index("s")

  pltpu.sync_copy(zero_hbm.at[pl.ds(0, ZR)], acc.at[pl.ds(s * ZR, ZR)])
  plsc.subcore_barrier()

  src_base = c * E_PAD + s * EPT
  dst_base = s * EPT

  @pl.loop(0, N_CHUNKS)
  def _body(g):
    pltpu.sync_copy(src_hbm.at[pl.ds(src_base + g * CHUNK, CHUNK)], sidx)
    pltpu.sync_copy(dst_hbm.at[pl.ds(dst_base + g * CHUNK, CHUNK)], didx)
    pltpu.async_copy(h2_hbm.at[sidx], rows, gsem).wait()
    pltpu.sync_copy(rows, acc.at[didx], add=True)

  plsc.subcore_barrier()

  pltpu.sync_copy(acc.at[pl.ds(s * ZR, ZR)],
                  out_hbm.at[pl.ds(c * ACC_ROWS + s * ZR, ZR)])


def _tc_matmul0(x, w0, deg2):
  def body(x_ref, w_ref, deg_ref, h2_ref):
    deg = deg_ref[0, :N, 0] + deg_ref[1, :N, 0] + 1.0
    dinv = lax.rsqrt(deg)
    h = jnp.dot(x_ref[...], w_ref[...], preferred_element_type=jnp.float32)
    hp = h * dinv[:, None]
    h2_ref[0, :, :] = hp[:, :HALF]
    h2_ref[1, :, :] = hp[:, HALF:]

  return pl.pallas_call(
      body,
      out_shape=jax.ShapeDtypeStruct((NC, N, HALF), jnp.float32),
  )(x, w0, deg2)


def _tc_layer(agg2, h2prev, deg2, b, g, be, w_next):
  final = w_next is None

  def body(agg_ref, hprev_ref, deg_ref, b_ref, g_ref, be_ref, *rest):
    if final:
      (out_ref,) = rest
    else:
      w_ref, out_ref = rest
    deg = deg_ref[0, :N, 0] + deg_ref[1, :N, 0] + 1.0
    dinv = lax.rsqrt(deg)

    halves = []
    for h_i in range(NC):
      z = (agg_ref[h_i, :N, :] + hprev_ref[h_i]) * dinv[:, None]
      z = z + b_ref[pl.ds(h_i * HALF, HALF)][None, :]
      mu = jnp.mean(z, axis=0)
      zc = z - mu[None, :]
      var = jnp.mean(zc * zc, axis=0)
      zn = (g_ref[pl.ds(h_i * HALF, HALF)][None, :] * zc
            * lax.rsqrt(var + EPS)[None, :]
            + be_ref[pl.ds(h_i * HALF, HALF)][None, :])
      halves.append(jnp.maximum(zn, 0.0))

    if final:
      out_ref[:, :HALF] = halves[0]
      out_ref[:, HALF:] = halves[1]
    else:
      hn = (jnp.dot(halves[0], w_ref[:HALF, :],
                    preferred_element_type=jnp.float32)
            + jnp.dot(halves[1], w_ref[HALF:, :],
                      preferred_element_type=jnp.float32))
      hp = hn * dinv[:, None]
      out_ref[0, :, :] = hp[:, :HALF]
      out_ref[1, :, :] = hp[:, HALF:]

  if final:
    out_shape = jax.ShapeDtypeStruct((N, D_H), jnp.float32)
    args = (agg2, h2prev, deg2, b, g, be)
  else:
    out_shape = jax.ShapeDtypeStruct((NC, N, HALF), jnp.float32)
    args = (agg2, h2prev, deg2, b, g, be, w_next)

  return pl.pallas_call(body, out_shape=out_shape)(*args)


def kernel(x, edge_index, W0, b0, g0, be0, W1, b1, g1, be1, W2, b2, g2, be2):
  src = edge_index[0]
  dst = edge_index[1]
  pad = E_PAD - E
  src_p = jnp.concatenate([src, jnp.zeros((pad,), jnp.int32)])
  dst_p = jnp.concatenate([dst, jnp.full((pad,), N, jnp.int32)])
  src2 = jnp.concatenate([src_p, src_p + N])

  zero16 = jnp.zeros((ZR, 16), jnp.float32)
  zeroH = jnp.zeros((ZR, HALF), jnp.float32)

  deg2 = _sc_degree(dst_p, zero16)

  h2 = _tc_matmul0(x, W0, deg2)
  params = [(b0, g0, be0, W1), (b1, g1, be1, W2), (b2, g2, be2, None)]
  for b, g, be, w_next in params:
    aggflat = _sc_aggregate(h2.reshape(NC * N, HALF), src2, dst_p, zeroH)
    h2 = _tc_layer(aggflat.reshape(NC, ACC_ROWS, HALF), h2, deg2, b, g, be,
                   w_next)
  return h2
